# Initial kernel scaffold; baseline (speedup 1.0000x reference)
import jax
import jax.numpy as jnp
from jax.experimental import pallas as pl


def kernel(x, edge_index, batch, W1, as1, ad1, b1, W2, as2, ad2, b2, fcW, fcb):
    raise NotImplementedError("write your pallas kernel here")


# speedup vs baseline: 4.6278x; 4.6278x over previous
import jax
import jax.numpy as jnp
from jax import lax
from jax.experimental import pallas as pl
from jax.experimental.pallas import tpu as pltpu
from jax.experimental.pallas import tpu_sc as plsc

N = 10000
E = 160000
IN = 256
HID = 256
HEADS = 4
OUT = 128
HC = HEADS * HID
NCHUNK = 8
CW = HC // NCHUNK
AW = 16
ARW = 128

NC = 2
NS = 16
NW = NC * NS
EP = 163840
EPT = EP // NW
KB = 128
NBLK = EPT // KB
RBIG = 640
RLAST = N - (NS - 1) * RBIG

_mesh = plsc.VectorSubcoreMesh(core_axis_name="c", subcore_axis_name="s")

_GDN = lax.GatherDimensionNumbers(
    offset_dims=(), collapsed_slice_dims=(0,), start_index_map=(0,))


def _splat(row, lane):
    return lax.gather(row, jnp.full((16, 1), lane, jnp.int32), _GDN, (1,),
                      mode=lax.GatherScatterMode.PROMISE_IN_BOUNDS)


def _rows_copy(sid, src, dst):
    @pl.when(sid < NS - 1)
    def _():
        pltpu.sync_copy(src.at[pl.ds(sid * RBIG, RBIG)],
                        dst.at[pl.ds(sid * RBIG, RBIG)])

    @pl.when(sid == NS - 1)
    def _():
        pltpu.sync_copy(src.at[pl.ds((NS - 1) * RBIG, RLAST)],
                        dst.at[pl.ds((NS - 1) * RBIG, RLAST)])


BN = 400


def _attn_logits(h, as_ref, ad_ref):
    h3 = h.reshape(BN, HEADS, HID)
    a_src = jnp.sum(h3 * as_ref[...][None], axis=-1)
    a_dst = jnp.sum(h3 * ad_ref[...][None], axis=-1)
    pad = jnp.zeros((BN, ARW - HEADS), jnp.float32)
    return (jnp.concatenate([a_src, pad], axis=-1),
            jnp.concatenate([a_dst, pad], axis=-1))


def _matmul_attn_body(x_ref, w_ref, as_ref, ad_ref, h_ref, s_ref, d_ref):
    h = jnp.dot(x_ref[...], w_ref[...], preferred_element_type=jnp.float32)
    for ci in range(NCHUNK):
        h_ref[ci] = h[:, ci * CW:(ci + 1) * CW]
    s_ref[...], d_ref[...] = _attn_logits(h, as_ref, ad_ref)


def _tc_in(x, W, as_r, ad_r):
    return pl.pallas_call(
        _matmul_attn_body,
        grid=(N // BN,),
        in_specs=[
            pl.BlockSpec((BN, IN), lambda i: (i, 0)),
            pl.BlockSpec((IN, HC), lambda i: (0, 0)),
            pl.BlockSpec((HEADS, HID), lambda i: (0, 0)),
            pl.BlockSpec((HEADS, HID), lambda i: (0, 0)),
        ],
        out_specs=[
            pl.BlockSpec((NCHUNK, BN, CW), lambda i: (0, i, 0)),
            pl.BlockSpec((BN, ARW), lambda i: (i, 0)),
            pl.BlockSpec((BN, ARW), lambda i: (i, 0)),
        ],
        out_shape=[
            jax.ShapeDtypeStruct((NCHUNK, N, CW), jnp.float32),
            jax.ShapeDtypeStruct((N, ARW), jnp.float32),
            jax.ShapeDtypeStruct((N, ARW), jnp.float32),
        ],
    )(x, W, as_r, ad_r)


def _combine(np_ref, dp_ref, b_ref):
    num = jnp.concatenate(
        [np_ref[0, ci] + np_ref[1, ci] for ci in range(NCHUNK)], axis=-1)
    den = jnp.max((dp_ref[0] + dp_ref[1]).reshape(BN, HEADS, 32), axis=-1)
    num3 = num.reshape(BN, HEADS, HID)
    den3 = jnp.maximum(den[:, :, None], 1e-30)
    x = (num3 / den3).reshape(BN, HC)
    x = x + b_ref[...][None]
    return jnp.maximum(x, 0.0) + jnp.exp(jnp.minimum(x, 0.0)) - 1.0


def _combine_matmul_attn_body(np_ref, dp_ref, b_ref, w_ref, as_ref, ad_ref,
                              h_ref, s_ref, d_ref):
    x = _combine(np_ref, dp_ref, b_ref)
    h = jnp.dot(x, w_ref[...], preferred_element_type=jnp.float32)
    for ci in range(NCHUNK):
        h_ref[ci] = h[:, ci * CW:(ci + 1) * CW]
    s_ref[...], d_ref[...] = _attn_logits(h, as_ref, ad_ref)


def _tc_mid(nump, denp, b, W, as_r, ad_r):
    return pl.pallas_call(
        _combine_matmul_attn_body,
        grid=(N // BN,),
        in_specs=[
            pl.BlockSpec((2, NCHUNK, BN, CW), lambda i: (0, 0, i, 0)),
            pl.BlockSpec((2, BN, CW), lambda i: (0, i, 0)),
            pl.BlockSpec((HC,), lambda i: (0,)),
            pl.BlockSpec((HC, HC), lambda i: (0, 0)),
            pl.BlockSpec((HEADS, HID), lambda i: (0, 0)),
            pl.BlockSpec((HEADS, HID), lambda i: (0, 0)),
        ],
        out_specs=[
            pl.BlockSpec((NCHUNK, BN, CW), lambda i: (0, i, 0)),
            pl.BlockSpec((BN, ARW), lambda i: (i, 0)),
            pl.BlockSpec((BN, ARW), lambda i: (i, 0)),
        ],
        out_shape=[
            jax.ShapeDtypeStruct((NCHUNK, N, CW), jnp.float32),
            jax.ShapeDtypeStruct((N, ARW), jnp.float32),
            jax.ShapeDtypeStruct((N, ARW), jnp.float32),
        ],
    )(nump, denp, b, W, as_r, ad_r)


def _combine_fc_body(np_ref, dp_ref, b_ref, w_ref, fb_ref, o_ref):
    x = _combine(np_ref, dp_ref, b_ref)
    o_ref[...] = (jnp.dot(x, w_ref[...], preferred_element_type=jnp.float32)
                  + fb_ref[...][None])


def _tc_out(nump, denp, b, fcW, fcb):
    return pl.pallas_call(
        _combine_fc_body,
        grid=(N // BN,),
        in_specs=[
            pl.BlockSpec((2, NCHUNK, BN, CW), lambda i: (0, 0, i, 0)),
            pl.BlockSpec((2, BN, CW), lambda i: (0, i, 0)),
            pl.BlockSpec((HC,), lambda i: (0,)),
            pl.BlockSpec((HC, OUT), lambda i: (0, 0)),
            pl.BlockSpec((OUT,), lambda i: (0,)),
        ],
        out_specs=pl.BlockSpec((BN, OUT), lambda i: (i, 0)),
        out_shape=jax.ShapeDtypeStruct((N, OUT), jnp.float32),
    )(nump, denp, b, fcW, fcb)


def _sc_a_body(srcp, dstp, aS, aD, w_out,
               src_si, dst_si, as_v, ad_v, w16_v, sem):
    cid = lax.axis_index("c")
    sid = lax.axis_index("s")
    wid = sid * NC + cid
    base = wid * EPT

    for k in range(NBLK):
        pltpu.sync_copy(srcp.at[pl.ds(base + k * KB, KB)], src_si.at[k])
        pltpu.sync_copy(dstp.at[pl.ds(base + k * KB, KB)], dst_si.at[k])

    @pl.loop(0, NBLK)
    def _blk(j):
        pltpu.async_copy(aS.at[src_si.at[j]], as_v, sem).wait()
        pltpu.async_copy(aD.at[dst_si.at[j]], ad_v, sem).wait()
        g0 = base + j * KB

        @pl.loop(0, KB, unroll=8)
        def _edge(e):
            s = as_v[e, pl.ds(0, 16)] + ad_v[e, pl.ds(0, 16)]
            s = jnp.maximum(s, 0.2 * s)
            w = jnp.exp(s)
            gv = jnp.full((16,), g0 + e, jnp.int32)
            mf = jnp.clip((E - gv).astype(jnp.float32), 0.0, 1.0)
            w16_v[pl.ds(e * AW, 16)] = w * mf

        pltpu.sync_copy(w16_v, w_out.at[pl.ds(g0 * AW, KB * AW)])


_sc_a = pl.kernel(
    _sc_a_body,
    out_type=jax.ShapeDtypeStruct((EP * AW,), jnp.float32),
    mesh=_mesh,
    scratch_types=[
        pltpu.VMEM((NBLK, KB), jnp.int32),
        pltpu.VMEM((NBLK, KB), jnp.int32),
        pltpu.VMEM((KB, ARW), jnp.float32),
        pltpu.VMEM((KB, ARW), jnp.float32),
        pltpu.VMEM((KB * AW,), jnp.float32),
        pltpu.SemaphoreType.DMA,
    ],
)


def _sc_b_body(srcp, dstp, w_hbm, hT, z128, num_out, den_out,
               src_si, dst_si, w_v, rows_v, num_sp, sem):
    cid = lax.axis_index("c")
    sid = lax.axis_index("s")
    wid = sid * NC + cid
    base = wid * EPT

    for k in range(NBLK):
        pltpu.sync_copy(srcp.at[pl.ds(base + k * KB, KB)], src_si.at[k])
        pltpu.sync_copy(dstp.at[pl.ds(base + k * KB, KB)], dst_si.at[k])

    for c in range(NCHUNK):
        head = c // 2
        _rows_copy(sid, z128, num_sp)
        plsc.subcore_barrier()

        @pl.loop(0, NBLK)
        def _blk(j):
            pltpu.async_copy(hT.at[c].at[src_si.at[j]], rows_v, sem).wait()
            pltpu.sync_copy(w_hbm.at[pl.ds((base + j * KB) * AW, KB * AW)],
                            w_v)

            @pl.loop(0, KB, unroll=4)
            def _edge(e):
                ws = _splat(w_v[pl.ds(e * AW, 16)], head)
                for v in range(CW // 16):
                    rows_v[e, pl.ds(v * 16, 16)] = (
                        rows_v[e, pl.ds(v * 16, 16)] * ws)

            pltpu.sync_copy(rows_v, num_sp.at[dst_si.at[j]], add=True)

        plsc.subcore_barrier()
        _rows_copy(sid, num_sp, num_out.at[cid, c])
        plsc.subcore_barrier()

    _rows_copy(sid, z128, num_sp)
    plsc.subcore_barrier()

    @pl.loop(0, NBLK)
    def _dblk(j):
        pltpu.sync_copy(w_hbm.at[pl.ds((base + j * KB) * AW, KB * AW)], w_v)

        @pl.loop(0, KB, unroll=4)
        def _edge(e):
            wrow = w_v[pl.ds(e * AW, 16)]
            for v in range(CW // 16):
                rows_v[e, pl.ds(v * 16, 16)] = (
                    rows_v[e, pl.ds(v * 16, 16)] * 0.0 + _splat(wrow, v // 2))

        pltpu.sync_copy(rows_v, num_sp.at[dst_si.at[j]], add=True)

    plsc.subcore_barrier()
    _rows_copy(sid, num_sp, den_out.at[cid])
    plsc.subcore_barrier()


_sc_b = pl.kernel(
    _sc_b_body,
    out_type=(
        jax.ShapeDtypeStruct((NC, NCHUNK, N, CW), jnp.float32),
        jax.ShapeDtypeStruct((NC, N, CW), jnp.float32),
    ),
    mesh=_mesh,
    scratch_types=[
        pltpu.VMEM((NBLK, KB), jnp.int32),
        pltpu.VMEM((NBLK, KB), jnp.int32),
        pltpu.VMEM((KB * AW,), jnp.float32),
        pltpu.VMEM((KB, CW), jnp.float32),
        pltpu.VMEM_SHARED((N, CW), jnp.float32),
        pltpu.SemaphoreType.DMA,
    ],
)


def kernel(x, edge_index, batch, W1, as1, ad1, b1, W2, as2, ad2, b2, fcW, fcb):
    del batch
    pad = EP - E
    srcp = jnp.concatenate([edge_index[0], jnp.zeros((pad,), jnp.int32)])
    dstp = jnp.concatenate([edge_index[1], jnp.zeros((pad,), jnp.int32)])
    z128 = jnp.zeros((N, CW), jnp.float32)
    as1r = as1.reshape(HEADS, HID)
    ad1r = ad1.reshape(HEADS, HID)
    as2r = as2.reshape(HEADS, HID)
    ad2r = ad2.reshape(HEADS, HID)

    hT1, aS1, aD1 = _tc_in(x, W1, as1r, ad1r)
    w1 = _sc_a(srcp, dstp, aS1, aD1)
    num1, den1 = _sc_b(srcp, dstp, w1, hT1, z128)
    hT2, aS2, aD2 = _tc_mid(num1, den1, b1, W2, as2r, ad2r)
    w2 = _sc_a(srcp, dstp, aS2, aD2)
    num2, den2 = _sc_b(srcp, dstp, w2, hT2, z128)
    return _tc_out(num2, den2, b2, fcW, fcb)


# speedup vs baseline: 6.2747x; 1.3559x over previous
import jax
import jax.numpy as jnp
from jax import lax
from jax.experimental import pallas as pl
from jax.experimental.pallas import tpu as pltpu
from jax.experimental.pallas import tpu_sc as plsc

N = 10000
E = 160000
IN = 256
HID = 256
HEADS = 4
OUT = 128
HC = HEADS * HID
NCHUNK = 8
CW = HC // NCHUNK
AW = 16
ARW = 128

NC = 2
NS = 16
NW = NC * NS
EP = 163840
EPT = EP // NW
KB = 128
NBLK = EPT // KB
RBIG = 640
RLAST = N - (NS - 1) * RBIG

_mesh = plsc.VectorSubcoreMesh(core_axis_name="c", subcore_axis_name="s")

_GDN = lax.GatherDimensionNumbers(
    offset_dims=(), collapsed_slice_dims=(0,), start_index_map=(0,))


def _splat(row, lane):
    return lax.gather(row, jnp.full((16, 1), lane, jnp.int32), _GDN, (1,),
                      mode=lax.GatherScatterMode.PROMISE_IN_BOUNDS)


def _rows_copy(sid, src, dst):
    @pl.when(sid < NS - 1)
    def _():
        pltpu.sync_copy(src.at[pl.ds(sid * RBIG, RBIG)],
                        dst.at[pl.ds(sid * RBIG, RBIG)])

    @pl.when(sid == NS - 1)
    def _():
        pltpu.sync_copy(src.at[pl.ds((NS - 1) * RBIG, RLAST)],
                        dst.at[pl.ds((NS - 1) * RBIG, RLAST)])


BN = 400


def _attn_logits(h, as_ref, ad_ref):
    h3 = h.reshape(BN, HEADS, HID)
    a_src = jnp.sum(h3 * as_ref[...][None], axis=-1)
    a_dst = jnp.sum(h3 * ad_ref[...][None], axis=-1)
    pad = jnp.zeros((BN, ARW - HEADS), jnp.float32)
    return (jnp.concatenate([a_src, pad], axis=-1),
            jnp.concatenate([a_dst, pad], axis=-1))


def _matmul_attn_body(x_ref, w_ref, as_ref, ad_ref, h_ref, s_ref, d_ref):
    h = jnp.dot(x_ref[...], w_ref[...], preferred_element_type=jnp.float32)
    for ci in range(NCHUNK):
        h_ref[ci] = h[:, ci * CW:(ci + 1) * CW]
    s_ref[...], d_ref[...] = _attn_logits(h, as_ref, ad_ref)


def _tc_in(x, W, as_r, ad_r):
    return pl.pallas_call(
        _matmul_attn_body,
        grid=(N // BN,),
        in_specs=[
            pl.BlockSpec((BN, IN), lambda i: (i, 0)),
            pl.BlockSpec((IN, HC), lambda i: (0, 0)),
            pl.BlockSpec((HEADS, HID), lambda i: (0, 0)),
            pl.BlockSpec((HEADS, HID), lambda i: (0, 0)),
        ],
        out_specs=[
            pl.BlockSpec((NCHUNK, BN, CW), lambda i: (0, i, 0)),
            pl.BlockSpec((BN, ARW), lambda i: (i, 0)),
            pl.BlockSpec((BN, ARW), lambda i: (i, 0)),
        ],
        out_shape=[
            jax.ShapeDtypeStruct((NCHUNK, N, CW), jnp.float32),
            jax.ShapeDtypeStruct((N, ARW), jnp.float32),
            jax.ShapeDtypeStruct((N, ARW), jnp.float32),
        ],
    )(x, W, as_r, ad_r)


def _combine(np_ref, dp_ref, b_ref):
    num = jnp.concatenate(
        [np_ref[0, ci] + np_ref[1, ci] for ci in range(NCHUNK)], axis=-1)
    den = jnp.max((dp_ref[0] + dp_ref[1]).reshape(BN, HEADS, 32), axis=-1)
    num3 = num.reshape(BN, HEADS, HID)
    den3 = jnp.maximum(den[:, :, None], 1e-30)
    x = (num3 / den3).reshape(BN, HC)
    x = x + b_ref[...][None]
    return jnp.maximum(x, 0.0) + jnp.exp(jnp.minimum(x, 0.0)) - 1.0


def _combine_matmul_attn_body(np_ref, dp_ref, b_ref, w_ref, as_ref, ad_ref,
                              h_ref, s_ref, d_ref):
    x = _combine(np_ref, dp_ref, b_ref)
    h = jnp.dot(x, w_ref[...], preferred_element_type=jnp.float32)
    for ci in range(NCHUNK):
        h_ref[ci] = h[:, ci * CW:(ci + 1) * CW]
    s_ref[...], d_ref[...] = _attn_logits(h, as_ref, ad_ref)


def _tc_mid(nump, denp, b, W, as_r, ad_r):
    return pl.pallas_call(
        _combine_matmul_attn_body,
        grid=(N // BN,),
        in_specs=[
            pl.BlockSpec((2, NCHUNK, BN, CW), lambda i: (0, 0, i, 0)),
            pl.BlockSpec((2, BN, CW), lambda i: (0, i, 0)),
            pl.BlockSpec((HC,), lambda i: (0,)),
            pl.BlockSpec((HC, HC), lambda i: (0, 0)),
            pl.BlockSpec((HEADS, HID), lambda i: (0, 0)),
            pl.BlockSpec((HEADS, HID), lambda i: (0, 0)),
        ],
        out_specs=[
            pl.BlockSpec((NCHUNK, BN, CW), lambda i: (0, i, 0)),
            pl.BlockSpec((BN, ARW), lambda i: (i, 0)),
            pl.BlockSpec((BN, ARW), lambda i: (i, 0)),
        ],
        out_shape=[
            jax.ShapeDtypeStruct((NCHUNK, N, CW), jnp.float32),
            jax.ShapeDtypeStruct((N, ARW), jnp.float32),
            jax.ShapeDtypeStruct((N, ARW), jnp.float32),
        ],
    )(nump, denp, b, W, as_r, ad_r)


def _combine_fc_body(np_ref, dp_ref, b_ref, w_ref, fb_ref, o_ref):
    x = _combine(np_ref, dp_ref, b_ref)
    o_ref[...] = (jnp.dot(x, w_ref[...], preferred_element_type=jnp.float32)
                  + fb_ref[...][None])


def _tc_out(nump, denp, b, fcW, fcb):
    return pl.pallas_call(
        _combine_fc_body,
        grid=(N // BN,),
        in_specs=[
            pl.BlockSpec((2, NCHUNK, BN, CW), lambda i: (0, 0, i, 0)),
            pl.BlockSpec((2, BN, CW), lambda i: (0, i, 0)),
            pl.BlockSpec((HC,), lambda i: (0,)),
            pl.BlockSpec((HC, OUT), lambda i: (0, 0)),
            pl.BlockSpec((OUT,), lambda i: (0,)),
        ],
        out_specs=pl.BlockSpec((BN, OUT), lambda i: (i, 0)),
        out_shape=jax.ShapeDtypeStruct((N, OUT), jnp.float32),
    )(nump, denp, b, fcW, fcb)


def _sc_a_body(srcp, dstp, aS, aD, w_out,
               src_si, dst_si, as_v, ad_v, w16_v, sem):
    cid = lax.axis_index("c")
    sid = lax.axis_index("s")
    wid = sid * NC + cid
    base = wid * EPT

    for k in range(NBLK):
        pltpu.sync_copy(srcp.at[pl.ds(base + k * KB, KB)], src_si.at[k])
        pltpu.sync_copy(dstp.at[pl.ds(base + k * KB, KB)], dst_si.at[k])

    @pl.loop(0, NBLK)
    def _blk(j):
        pltpu.async_copy(aS.at[src_si.at[j]], as_v, sem).wait()
        pltpu.async_copy(aD.at[dst_si.at[j]], ad_v, sem).wait()
        g0 = base + j * KB

        @pl.loop(0, KB, unroll=8)
        def _edge(e):
            s = as_v[e, pl.ds(0, 16)] + ad_v[e, pl.ds(0, 16)]
            s = jnp.maximum(s, 0.2 * s)
            w = jnp.exp(s)
            gv = jnp.full((16,), g0 + e, jnp.int32)
            mf = jnp.clip((E - gv).astype(jnp.float32), 0.0, 1.0)
            w16_v[pl.ds(e * AW, 16)] = w * mf

        pltpu.sync_copy(w16_v, w_out.at[pl.ds(g0 * AW, KB * AW)])


_sc_a = pl.kernel(
    _sc_a_body,
    out_type=jax.ShapeDtypeStruct((EP * AW,), jnp.float32),
    mesh=_mesh,
    scratch_types=[
        pltpu.VMEM((NBLK, KB), jnp.int32),
        pltpu.VMEM((NBLK, KB), jnp.int32),
        pltpu.VMEM((KB, ARW), jnp.float32),
        pltpu.VMEM((KB, ARW), jnp.float32),
        pltpu.VMEM((KB * AW,), jnp.float32),
        pltpu.SemaphoreType.DMA,
    ],
)


def _sc_b_body(srcp, dstp, w_hbm, hT, z128, num_out, den_out,
               src_si, dst_si, w_blk, rows_a, rows_b, num_sp, sem_a, sem_b):
    cid = lax.axis_index("c")
    sid = lax.axis_index("s")
    wid = sid * NC + cid
    base = wid * EPT

    for k in range(NBLK):
        pltpu.sync_copy(srcp.at[pl.ds(base + k * KB, KB)], src_si.at[k])
        pltpu.sync_copy(dstp.at[pl.ds(base + k * KB, KB)], dst_si.at[k])
    def _scale_scatter(rows_v, j, head):
        pltpu.sync_copy(w_hbm.at[pl.ds((base + j * KB) * AW, KB * AW)], w_blk)

        @pl.loop(0, KB, unroll=4)
        def _edge(e):
            ws = _splat(w_blk[pl.ds(e * AW, 16)], head)
            for v in range(CW // 16):
                rows_v[e, pl.ds(v * 16, 16)] = (
                    rows_v[e, pl.ds(v * 16, 16)] * ws)

        pltpu.sync_copy(rows_v, num_sp.at[dst_si.at[j]], add=True)

    for c in range(NCHUNK):
        head = c // 2
        _rows_copy(sid, z128, num_sp)
        plsc.subcore_barrier()

        pltpu.async_copy(hT.at[c].at[src_si.at[0]], rows_a, sem_a)

        @pl.loop(0, NBLK // 2)
        def _pair(j2):
            j0 = j2 * 2
            pltpu.async_copy(hT.at[c].at[src_si.at[j0 + 1]], rows_b, sem_b)
            pltpu.make_async_copy(hT.at[c].at[src_si.at[j0]],
                                  rows_a, sem_a).wait()
            _scale_scatter(rows_a, j0, head)

            @pl.when(j2 < NBLK // 2 - 1)
            def _():
                pltpu.async_copy(hT.at[c].at[src_si.at[j0 + 2]],
                                 rows_a, sem_a)

            pltpu.make_async_copy(hT.at[c].at[src_si.at[j0 + 1]],
                                  rows_b, sem_b).wait()
            _scale_scatter(rows_b, j0 + 1, head)

        plsc.subcore_barrier()
        _rows_copy(sid, num_sp, num_out.at[cid, c])
        plsc.subcore_barrier()

    _rows_copy(sid, z128, num_sp)
    plsc.subcore_barrier()

    @pl.loop(0, NBLK)
    def _dblk(j):
        pltpu.sync_copy(w_hbm.at[pl.ds((base + j * KB) * AW, KB * AW)], w_blk)

        @pl.loop(0, KB, unroll=4)
        def _edge(e):
            wrow = w_blk[pl.ds(e * AW, 16)]
            for v in range(CW // 16):
                rows_a[e, pl.ds(v * 16, 16)] = (
                    rows_a[e, pl.ds(v * 16, 16)] * 0.0 + _splat(wrow, v // 2))

        pltpu.sync_copy(rows_a, num_sp.at[dst_si.at[j]], add=True)

    plsc.subcore_barrier()
    _rows_copy(sid, num_sp, den_out.at[cid])
    plsc.subcore_barrier()


_sc_b = pl.kernel(
    _sc_b_body,
    out_type=(
        jax.ShapeDtypeStruct((NC, NCHUNK, N, CW), jnp.float32),
        jax.ShapeDtypeStruct((NC, N, CW), jnp.float32),
    ),
    mesh=_mesh,
    scratch_types=[
        pltpu.VMEM((NBLK, KB), jnp.int32),
        pltpu.VMEM((NBLK, KB), jnp.int32),
        pltpu.VMEM((KB * AW,), jnp.float32),
        pltpu.VMEM((KB, CW), jnp.float32),
        pltpu.VMEM((KB, CW), jnp.float32),
        pltpu.VMEM_SHARED((N, CW), jnp.float32),
        pltpu.SemaphoreType.DMA,
        pltpu.SemaphoreType.DMA,
    ],
)


def kernel(x, edge_index, batch, W1, as1, ad1, b1, W2, as2, ad2, b2, fcW, fcb):
    del batch
    pad = EP - E
    srcp = jnp.concatenate([edge_index[0], jnp.zeros((pad,), jnp.int32)])
    dstp = jnp.concatenate([edge_index[1], jnp.zeros((pad,), jnp.int32)])
    z128 = jnp.zeros((N, CW), jnp.float32)
    as1r = as1.reshape(HEADS, HID)
    ad1r = ad1.reshape(HEADS, HID)
    as2r = as2.reshape(HEADS, HID)
    ad2r = ad2.reshape(HEADS, HID)

    hT1, aS1, aD1 = _tc_in(x, W1, as1r, ad1r)
    w1 = _sc_a(srcp, dstp, aS1, aD1)
    num1, den1 = _sc_b(srcp, dstp, w1, hT1, z128)
    hT2, aS2, aD2 = _tc_mid(num1, den1, b1, W2, as2r, ad2r)
    w2 = _sc_a(srcp, dstp, aS2, aD2)
    num2, den2 = _sc_b(srcp, dstp, w2, hT2, z128)
    return _tc_out(num2, den2, b2, fcW, fcb)
